# Initial kernel scaffold; baseline (speedup 1.0000x reference)
import jax
import jax.numpy as jnp
from jax.experimental import pallas as pl


def kernel(embeddings, table):
    raise NotImplementedError("write your pallas kernel here")


# speedup vs baseline: 2.9425x; 2.9425x over previous
import functools

import jax
import jax.numpy as jnp
from jax import lax
from jax.experimental import pallas as pl
from jax.experimental.pallas import tpu as pltpu
from jax.experimental.pallas import tpu_sc as plsc

_D = 1024
_S = 4096
_B = 4
_NC = 2
_NS = 16
_NW = _NC * _NS
_ROWS_PER_W = _S // _NW
_CH = 64

_mesh = plsc.VectorSubcoreMesh(
    core_axis_name="c", subcore_axis_name="s", num_cores=_NC, num_subcores=_NS
)


@functools.partial(
    pl.kernel,
    mesh=_mesh,
    out_type=jax.ShapeDtypeStruct((_B, _S, _D), jnp.float32),
    scratch_types=[
        pltpu.VMEM((_CH, _D), jnp.float32),
        pltpu.SemaphoreType.DMA,
    ],
)
def _pos_embed_sc(table_hbm, out_hbm, buf, sem):
    wid = lax.axis_index("s") * _NC + lax.axis_index("c")
    base = wid * _ROWS_PER_W
    for p in range(_ROWS_PER_W // _CH):
        off = base + p * _CH
        pltpu.sync_copy(table_hbm.at[pl.ds(off, _CH)], buf)
        for b in range(_B):
            pltpu.sync_copy(buf, out_hbm.at[b, pl.ds(off, _CH)])


def kernel(embeddings, table):
    del embeddings
    return _pos_embed_sc(table)
